# baseline (speedup 1.0000x reference)
import functools

import jax
import jax.numpy as jnp
from jax import lax
from jax.experimental import pallas as pl
from jax.experimental.pallas import tpu as pltpu
from jax.experimental.pallas import tpu_sc as plsc

_NC = 2
_NS = 16
_CH = 88
_NB = 4
_HALF = 128


def _sc_scatter_prop(src, rows_t, cols_t, ztile, S, NPAD, NCHUNK, RPT):
  mesh = plsc.VectorSubcoreMesh(core_axis_name="c", subcore_axis_name="s")

  @functools.partial(
      pl.kernel,
      out_type=jax.ShapeDtypeStruct((S, _NC, NPAD, _HALF), jnp.float32),
      mesh=mesh,
      scratch_types=(
          [pltpu.VMEM((_NB, _CH), jnp.int32),
           pltpu.VMEM((_NB, _CH), jnp.int32)] +
          [pltpu.VMEM((_CH, _HALF), jnp.float32)
           for _ in range(_NB)] +
          [pltpu.VMEM_SHARED((NPAD, _HALF), jnp.float32)] +
          [pltpu.SemaphoreType.DMA] * (4 * _NB)
      ),
  )
  def k(src_hbm, rows_hbm, cols_hbm, z_hbm, out_hbm,
        ricb, cicb, *rest):
    gbufs = rest[:_NB]
    acc = rest[_NB]
    sems = rest[_NB + 1:]
    gsems = sems[:_NB]
    ssems = sems[_NB:2 * _NB]
    risems = sems[2 * _NB:3 * _NB]
    cisems = sems[3 * _NB:]

    cid = lax.axis_index("c")
    tid = lax.axis_index("s")

    stripe = pl.ds(tid * RPT, RPT)
    for s in range(S):
      tab = src_hbm.at[s, cid]
      pltpu.sync_copy(z_hbm, acc.at[stripe])
      for q in range(_NB):
        pltpu.async_copy(rows_hbm.at[tid, q], ricb.at[q], risems[q])
      for q in range(2):
        pltpu.async_copy(cols_hbm.at[tid, q], cicb.at[q], cisems[q])
      for b in range(2):
        pltpu.make_async_copy(rows_hbm.at[tid, b], ricb.at[b],
                              risems[b]).wait()
        pltpu.async_copy(tab.at[ricb.at[b]], gbufs[b], gsems[b])
      plsc.subcore_barrier()

      def body(i, carry):
        for j in range(_NB):
          kk = _NB * i + j
          f = (j + 2) % _NB
          pltpu.make_async_copy(tab.at[ricb.at[j]], gbufs[j],
                                gsems[j]).wait()
          @pl.when(kk + _NB < NCHUNK)
          def _():
            pltpu.async_copy(rows_hbm.at[tid, kk + _NB], ricb.at[j],
                             risems[j])
          pltpu.make_async_copy(cols_hbm.at[tid, kk], cicb.at[j],
                                cisems[j]).wait()
          pltpu.async_copy(gbufs[j], acc.at[cicb.at[j]], ssems[j],
                           add=True)
          @pl.when(kk >= 2)
          def _():
            pltpu.make_async_copy(gbufs[f], acc.at[cicb.at[f]],
                                  ssems[f]).wait()
          @pl.when(kk + 2 < NCHUNK)
          def _():
            pltpu.async_copy(cols_hbm.at[tid, kk + 2], cicb.at[f],
                             cisems[f])
            pltpu.make_async_copy(rows_hbm.at[tid, kk + 2], ricb.at[f],
                                  risems[f]).wait()
            pltpu.async_copy(tab.at[ricb.at[f]], gbufs[f], gsems[f])
        return carry

      lax.fori_loop(0, NCHUNK // _NB, body, 0)

      for kk in (NCHUNK - 2, NCHUNK - 1):
        b = kk % _NB
        pltpu.make_async_copy(gbufs[b], acc.at[cicb.at[b]],
                              ssems[b]).wait()

      plsc.subcore_barrier()
      pltpu.sync_copy(acc.at[stripe], out_hbm.at[s, cid, stripe])

  return k(src, rows_t, cols_t, ztile)


def _tc_conv(xs, z1, z2, disr, wa, wb, wc, S, N, C, NPAD, RB, R):
  def body(x_ref, z1a_ref, z1b_ref, z2a_ref, z2b_ref, d_ref,
           wa_ref, wb_ref, wc_ref, o_ref, ps_ref, pq_ref):
    d = d_ref[:, 0:1]
    z1c = jnp.concatenate([z1a_ref[0, 0], z1b_ref[0, 0]], axis=-1) * d
    z2c = jnp.concatenate([z2a_ref[0, 0], z2b_ref[0, 0]], axis=-1) * d
    o = jnp.dot(x_ref[0], wa_ref[...], preferred_element_type=jnp.float32)
    o += jnp.dot(z1c, wb_ref[...], preferred_element_type=jnp.float32)
    o += jnp.dot(z2c, wc_ref[...], preferred_element_type=jnp.float32)
    o_ref[0] = o
    ps_ref[0, 0] = jnp.broadcast_to(jnp.sum(o, axis=0), (8, o.shape[-1]))
    pq_ref[0, 0] = jnp.broadcast_to(jnp.sum(o * o, axis=0), (8, o.shape[-1]))

  grid = (S, R)
  return pl.pallas_call(
      body,
      grid=grid,
      in_specs=[
          pl.BlockSpec((1, RB, C), lambda s, r: (s, r, 0)),
          pl.BlockSpec((1, 1, RB, _HALF), lambda s, r: (s, 0, r, 0)),
          pl.BlockSpec((1, 1, RB, _HALF), lambda s, r: (s, 1, r, 0)),
          pl.BlockSpec((1, 1, RB, _HALF), lambda s, r: (s, 0, r, 0)),
          pl.BlockSpec((1, 1, RB, _HALF), lambda s, r: (s, 1, r, 0)),
          pl.BlockSpec((RB, 8), lambda s, r: (r, 0)),
          pl.BlockSpec((C, C), lambda s, r: (0, 0)),
          pl.BlockSpec((C, C), lambda s, r: (0, 0)),
          pl.BlockSpec((C, C), lambda s, r: (0, 0)),
      ],
      out_specs=[
          pl.BlockSpec((1, RB, C), lambda s, r: (s, r, 0)),
          pl.BlockSpec((1, 1, 8, C), lambda s, r: (s, r, 0, 0)),
          pl.BlockSpec((1, 1, 8, C), lambda s, r: (s, r, 0, 0)),
      ],
      out_shape=[
          jax.ShapeDtypeStruct((S, N, C), jnp.float32),
          jax.ShapeDtypeStruct((S, R, 8, C), jnp.float32),
          jax.ShapeDtypeStruct((S, R, 8, C), jnp.float32),
      ],
  )(xs, z1, z1, z2, z2, disr, wa, wb, wc)


def _tc_bn_relu(conv, psum, psq, gamma2, beta2, S, N, C, RB, R):
  def body(o_ref, ps_ref, pq_ref, g_ref, b_ref, y_ref):
    inv_n = 1.0 / N
    mu = jnp.sum(ps_ref[0, :, 0, :], axis=0) * inv_n
    ex2 = jnp.sum(pq_ref[0, :, 0, :], axis=0) * inv_n
    var = ex2 - mu * mu
    scale = g_ref[0] * lax.rsqrt(var + 1e-5)
    y_ref[0] = jnp.maximum((o_ref[0] - mu) * scale + b_ref[0], 0.0)

  return pl.pallas_call(
      body,
      grid=(S, R),
      in_specs=[
          pl.BlockSpec((1, RB, C), lambda s, r: (s, r, 0)),
          pl.BlockSpec((1, R, 8, C), lambda s, r: (s, 0, 0, 0)),
          pl.BlockSpec((1, R, 8, C), lambda s, r: (s, 0, 0, 0)),
          pl.BlockSpec((1, C), lambda s, r: (0, 0)),
          pl.BlockSpec((1, C), lambda s, r: (0, 0)),
      ],
      out_specs=pl.BlockSpec((1, RB, C), lambda s, r: (s, r, 0)),
      out_shape=jax.ShapeDtypeStruct((S, N, C), jnp.float32),
  )(conv, psum, psq, gamma2, beta2)


def kernel(x, edge_index, W, bias, gamma, beta):
  B, T, N, C = x.shape
  E = edge_index.shape[1]
  S = B * T
  del bias

  NPAD = ((N + 1 + 8 * _NS - 1) // (8 * _NS)) * (8 * _NS)
  RPT = NPAD // _NS
  ept = -(-E // _NS)
  NCHUNK = ((-(-ept // _CH) + 3) // 4) * 4
  EP = _NS * NCHUNK * _CH

  row = edge_index[0]
  col = edge_index[1]

  deg = jnp.zeros((N,), jnp.float32).at[row].add(1.0)
  dis = jnp.where(deg > 0, lax.rsqrt(deg), 0.0)
  disp = jnp.pad(dis, (0, NPAD - N))

  pad_ids = jnp.full((EP,), N, jnp.int32)
  rows_t = pad_ids.at[:E].set(row).reshape(_NS, NCHUNK, _CH)
  cols_t = pad_ids.at[:E].set(col).reshape(_NS, NCHUNK, _CH)
  ztile = jnp.zeros((RPT, _HALF), jnp.float32)

  xs = x.reshape(S, N, C)
  u0 = dis[None, :, None] * xs
  u0p = jnp.pad(u0, ((0, 0), (0, NPAD - N), (0, 0)))
  u0p = u0p.reshape(S, NPAD, _NC, _HALF).transpose(0, 2, 1, 3)

  SH = S // 2
  dis2 = (disp * disp)[None, None, :, None]
  z1_h, z2_h = [], []
  for h in range(2):
    z1h = _sc_scatter_prop(u0p[h * SH:(h + 1) * SH], rows_t, cols_t,
                           ztile, SH, NPAD, NCHUNK, RPT)
    z1_h.append(z1h)
  for h in range(2):
    z2_h.append(_sc_scatter_prop(z1_h[h] * dis2, rows_t, cols_t,
                                 ztile, SH, NPAD, NCHUNK, RPT))
  z1 = jnp.concatenate(z1_h, axis=0)
  z2 = jnp.concatenate(z2_h, axis=0)

  wa = W[0] - W[2]
  wb = -W[1]
  wc = 2.0 * W[2]
  disr = jnp.broadcast_to(dis[:, None], (N, 8))

  RB = 1000
  R = N // RB
  conv, psum, psq = _tc_conv(xs, z1, z2, disr, wa, wb, wc,
                             S, N, C, NPAD, RB, R)
  y = _tc_bn_relu(conv, psum, psq, gamma[None, :], beta[None, :],
                  S, N, C, RB, R)
  return y.reshape(B, T, N, C)


# speedup vs baseline: 1.5848x; 1.5848x over previous
import functools

import jax
import jax.numpy as jnp
from jax import lax
from jax.experimental import pallas as pl
from jax.experimental.pallas import tpu as pltpu
from jax.experimental.pallas import tpu_sc as plsc

_NC = 2
_NS = 16
_CH = 80
_NB = 3
_HALF = 128


def _sc_scatter_prop(src, rows_t, cols_t, ztile, S, NPAD, NCHUNK, RPT):
  mesh = plsc.VectorSubcoreMesh(core_axis_name="c", subcore_axis_name="s")

  @functools.partial(
      pl.kernel,
      out_type=jax.ShapeDtypeStruct((S, _NC, NPAD, _HALF), jnp.float32),
      mesh=mesh,
      scratch_types=(
          [pltpu.VMEM((_NB, _CH), jnp.int32),
           pltpu.VMEM((NCHUNK, _CH), jnp.int32)] +
          [pltpu.VMEM((_CH, _HALF), jnp.float32)
           for _ in range(_NB)] +
          [pltpu.VMEM_SHARED((NPAD, _HALF), jnp.float32)] +
          [pltpu.SemaphoreType.DMA] * (3 * _NB)
      ),
  )
  def k(src_hbm, rows_hbm, cols_hbm, z_hbm, out_hbm,
        ricb, colbuf, *rest):
    gbufs = rest[:_NB]
    acc = rest[_NB]
    sems = rest[_NB + 1:]
    gsems = sems[:_NB]
    ssems = sems[_NB:2 * _NB]
    risems = sems[2 * _NB:]

    cid = lax.axis_index("c")
    tid = lax.axis_index("s")

    pltpu.sync_copy(cols_hbm.at[tid], colbuf)

    stripe = pl.ds(tid * RPT, RPT)
    for s in range(S):
      tab = src_hbm.at[s, cid]
      pltpu.sync_copy(z_hbm, acc.at[stripe])
      for q in range(_NB):
        pltpu.async_copy(rows_hbm.at[tid, q], ricb.at[q], risems[q])
      for b in range(2):
        pltpu.make_async_copy(rows_hbm.at[tid, b], ricb.at[b],
                              risems[b]).wait()
        pltpu.async_copy(tab.at[ricb.at[b]], gbufs[b], gsems[b])
        pltpu.async_copy(rows_hbm.at[tid, b + _NB], ricb.at[b],
                         risems[b])
      plsc.subcore_barrier()

      def body(i, carry):
        for j in range(_NB):
          kk = _NB * i + j
          f = (j + 2) % _NB
          pltpu.make_async_copy(tab.at[ricb.at[j]], gbufs[j],
                                gsems[j]).wait()
          pltpu.async_copy(gbufs[j], acc.at[colbuf.at[kk]], ssems[j],
                           add=True)
          @pl.when(kk >= 1)
          def _():
            pltpu.make_async_copy(gbufs[f], acc.at[colbuf.at[kk - 1]],
                                  ssems[f]).wait()
          @pl.when(kk + 2 < NCHUNK)
          def _():
            pltpu.make_async_copy(rows_hbm.at[tid, kk + 2], ricb.at[f],
                                  risems[f]).wait()
            pltpu.async_copy(tab.at[ricb.at[f]], gbufs[f], gsems[f])

            @pl.when(kk + _NB + 2 < NCHUNK)
            def _():
              pltpu.async_copy(rows_hbm.at[tid, kk + _NB + 2],
                               ricb.at[f], risems[f])
        return carry

      lax.fori_loop(0, NCHUNK // _NB, body, 0)

      b = (NCHUNK - 1) % _NB
      pltpu.make_async_copy(gbufs[b], acc.at[colbuf.at[NCHUNK - 1]],
                            ssems[b]).wait()

      plsc.subcore_barrier()
      pltpu.sync_copy(acc.at[stripe], out_hbm.at[s, cid, stripe])

  return k(src, rows_t, cols_t, ztile)


def _tc_conv(xs, z1, z2, disr, wa, wb, wc, S, N, C, NPAD, RB, R):
  def body(x_ref, z1a_ref, z1b_ref, z2a_ref, z2b_ref, d_ref,
           wa_ref, wb_ref, wc_ref, o_ref, ps_ref, pq_ref):
    d = d_ref[:, 0:1]
    z1c = jnp.concatenate([z1a_ref[0, 0], z1b_ref[0, 0]], axis=-1) * d
    z2c = jnp.concatenate([z2a_ref[0, 0], z2b_ref[0, 0]], axis=-1) * d
    o = jnp.dot(x_ref[0], wa_ref[...], preferred_element_type=jnp.float32)
    o += jnp.dot(z1c, wb_ref[...], preferred_element_type=jnp.float32)
    o += jnp.dot(z2c, wc_ref[...], preferred_element_type=jnp.float32)
    o_ref[0] = o
    ps_ref[0, 0] = jnp.broadcast_to(jnp.sum(o, axis=0), (8, o.shape[-1]))
    pq_ref[0, 0] = jnp.broadcast_to(jnp.sum(o * o, axis=0), (8, o.shape[-1]))

  grid = (S, R)
  return pl.pallas_call(
      body,
      grid=grid,
      in_specs=[
          pl.BlockSpec((1, RB, C), lambda s, r: (s, r, 0)),
          pl.BlockSpec((1, 1, RB, _HALF), lambda s, r: (s, 0, r, 0)),
          pl.BlockSpec((1, 1, RB, _HALF), lambda s, r: (s, 1, r, 0)),
          pl.BlockSpec((1, 1, RB, _HALF), lambda s, r: (s, 0, r, 0)),
          pl.BlockSpec((1, 1, RB, _HALF), lambda s, r: (s, 1, r, 0)),
          pl.BlockSpec((RB, 8), lambda s, r: (r, 0)),
          pl.BlockSpec((C, C), lambda s, r: (0, 0)),
          pl.BlockSpec((C, C), lambda s, r: (0, 0)),
          pl.BlockSpec((C, C), lambda s, r: (0, 0)),
      ],
      out_specs=[
          pl.BlockSpec((1, RB, C), lambda s, r: (s, r, 0)),
          pl.BlockSpec((1, 1, 8, C), lambda s, r: (s, r, 0, 0)),
          pl.BlockSpec((1, 1, 8, C), lambda s, r: (s, r, 0, 0)),
      ],
      out_shape=[
          jax.ShapeDtypeStruct((S, N, C), jnp.float32),
          jax.ShapeDtypeStruct((S, R, 8, C), jnp.float32),
          jax.ShapeDtypeStruct((S, R, 8, C), jnp.float32),
      ],
  )(xs, z1, z1, z2, z2, disr, wa, wb, wc)


def _tc_bn_relu(conv, psum, psq, gamma2, beta2, S, N, C, RB, R):
  def body(o_ref, ps_ref, pq_ref, g_ref, b_ref, y_ref):
    inv_n = 1.0 / N
    mu = jnp.sum(ps_ref[0, :, 0, :], axis=0) * inv_n
    ex2 = jnp.sum(pq_ref[0, :, 0, :], axis=0) * inv_n
    var = ex2 - mu * mu
    scale = g_ref[0] * lax.rsqrt(var + 1e-5)
    y_ref[0] = jnp.maximum((o_ref[0] - mu) * scale + b_ref[0], 0.0)

  return pl.pallas_call(
      body,
      grid=(S, R),
      in_specs=[
          pl.BlockSpec((1, RB, C), lambda s, r: (s, r, 0)),
          pl.BlockSpec((1, R, 8, C), lambda s, r: (s, 0, 0, 0)),
          pl.BlockSpec((1, R, 8, C), lambda s, r: (s, 0, 0, 0)),
          pl.BlockSpec((1, C), lambda s, r: (0, 0)),
          pl.BlockSpec((1, C), lambda s, r: (0, 0)),
      ],
      out_specs=pl.BlockSpec((1, RB, C), lambda s, r: (s, r, 0)),
      out_shape=jax.ShapeDtypeStruct((S, N, C), jnp.float32),
  )(conv, psum, psq, gamma2, beta2)


def kernel(x, edge_index, W, bias, gamma, beta):
  B, T, N, C = x.shape
  E = edge_index.shape[1]
  S = B * T
  del bias

  NPAD = ((N + 1 + 8 * _NS - 1) // (8 * _NS)) * (8 * _NS)
  RPT = NPAD // _NS
  ept = -(-E // _NS)
  NCHUNK = ((-(-ept // _CH) + _NB - 1) // _NB) * _NB
  EP = _NS * NCHUNK * _CH

  row = edge_index[0]
  col = edge_index[1]

  deg = jnp.zeros((N,), jnp.float32).at[row].add(1.0)
  dis = jnp.where(deg > 0, lax.rsqrt(deg), 0.0)
  disp = jnp.pad(dis, (0, NPAD - N))

  pad_ids = jnp.full((EP,), N, jnp.int32)
  rows_t = pad_ids.at[:E].set(row).reshape(_NS, NCHUNK, _CH)
  cols_t = pad_ids.at[:E].set(col).reshape(_NS, NCHUNK, _CH)
  ztile = jnp.zeros((RPT, _HALF), jnp.float32)

  xs = x.reshape(S, N, C)
  u0 = dis[None, :, None] * xs
  u0p = jnp.pad(u0, ((0, 0), (0, NPAD - N), (0, 0)))
  u0p = u0p.reshape(S, NPAD, _NC, _HALF).transpose(0, 2, 1, 3)
  z1 = _sc_scatter_prop(u0p, rows_t, cols_t, ztile, S, NPAD, NCHUNK, RPT)

  u1 = z1 * (disp * disp)[None, None, :, None]
  z2 = _sc_scatter_prop(u1, rows_t, cols_t, ztile, S, NPAD, NCHUNK, RPT)

  wa = W[0] - W[2]
  wb = -W[1]
  wc = 2.0 * W[2]
  disr = jnp.broadcast_to(dis[:, None], (N, 8))

  RB = 1000
  R = N // RB
  conv, psum, psq = _tc_conv(xs, z1, z2, disr, wa, wb, wc,
                             S, N, C, NPAD, RB, R)
  y = _tc_bn_relu(conv, psum, psq, gamma[None, :], beta[None, :],
                  S, N, C, RB, R)
  return y.reshape(B, T, N, C)
